# baseline (speedup 1.0000x reference)
import functools
import math

import jax
import jax.numpy as jnp
from jax import lax
from jax.experimental import pallas as pl
from jax.experimental.pallas import tpu as pltpu
from jax.experimental.pallas import tpu_sc as plsc

D = 64
EPS = 1e-6
SQRT_D = math.sqrt(D)

NC = 2
NS = 16
NW = NC * NS

L = 16
NQ = D // 8


def _fused_kernel(table_p, x, pos8, gamma, beta):
    B, S = x.shape
    BPW = B // NW
    mesh = plsc.VectorSubcoreMesh(core_axis_name="c", subcore_axis_name="s")

    @functools.partial(
        pl.kernel,
        out_type=jax.ShapeDtypeStruct((S, NQ, NW, 8, BPW), jnp.float32),
        mesh=mesh,
        scratch_types=[
            pltpu.VMEM((S, BPW), jnp.int32),
            pltpu.VMEM((L, S), jnp.int32),
            pltpu.VMEM((2, BPW, D), jnp.float32),
            pltpu.VMEM((S, D), jnp.float32),
            pltpu.VMEM((2, D), jnp.float32),
            pltpu.VMEM((2, D, BPW + 1), jnp.float32),
            pltpu.SemaphoreType.DMA,
            pltpu.SemaphoreType.DMA,
        ],
        compiler_params=pltpu.CompilerParams(
            use_tc_tiling_on_sc=False, needs_layout_passes=False
        ),
    )
    def body(table_h, x_h, pos_h, gam_h, bet_h, out_h, idxt_v, tmp_v, slab_v,
             pos_v, gb_v, ob_v, gsem, osem):
        wid = lax.axis_index("s") * NC + lax.axis_index("c")
        b0 = wid * BPW
        pltpu.sync_copy(pos_h, pos_v)
        pltpu.sync_copy(gam_h, gb_v.at[0])
        pltpu.sync_copy(bet_h, gb_v.at[1])
        iota = lax.iota(jnp.int32, L)

        for ch in range(BPW // L):
            pltpu.sync_copy(x_h.at[pl.ds(b0 + ch * L, L)], tmp_v)

            def tr_body(s, carry, _ch=ch):
                sv = jnp.full((L,), s, jnp.int32)
                v = plsc.load_gather(tmp_v, [iota, sv])
                plsc.store_scatter(idxt_v, [sv, iota + _ch * L], v)
                return carry

            lax.fori_loop(0, S, tr_body, 0)

        def fire_gather(s, slab):
            pltpu.async_copy(
                table_h.at[idxt_v.at[s]], slab_v.at[slab], gsem)

        def wait_gather(slab):
            pltpu.make_async_copy(
                table_h.at[pl.ds(0, BPW)], slab_v.at[slab], gsem).wait()

        def fire_outs(s, slab):
            for q in range(NQ):
                pltpu.async_copy(
                    ob_v.at[slab, pl.ds(q * 8, 8), pl.ds(0, BPW)],
                    out_h.at[s, q, wid], osem)

        def wait_outs(slab):
            pltpu.make_async_copy(
                table_h.at[pl.ds(0, BPW)], slab_v.at[0], osem).wait()

        inv_d = 1.0 / D
        inv_dm1 = 1.0 / (D - 1)
        nv = D // L
        gvec = [gb_v[0, pl.ds(L * k, L)] for k in range(nv)]
        bvec = [gb_v[1, pl.ds(L * k, L)] for k in range(nv)]
        crows = [iota + L * k for k in range(nv)]

        def compute(s, slab):
            pvec = [pos_v[s, pl.ds(L * k, L)] for k in range(nv)]

            def tok4(ti, carry, _slab=slab, _pvec=pvec):
                toks = []
                for u in range(8):
                    t = ti * 8 + u
                    h = [
                        slab_v[_slab, t, pl.ds(L * k, L)] + _pvec[k]
                        for k in range(nv)
                    ]
                    toks.append((t, h))
                stats = []
                for t, h in toks:
                    hs = (h[0] + h[1]) + (h[2] + h[3])
                    hq = (h[0] * h[0] + h[1] * h[1]) + (
                        h[2] * h[2] + h[3] * h[3])
                    stats.append((jnp.sum(hs), jnp.sum(hq)))
                for (t, h), (sm, sq) in zip(toks, stats):
                    mean = sm * inv_d
                    var = jnp.maximum((sq - sm * mean) * inv_dm1, 1e-30)
                    bits = lax.bitcast_convert_type(var, jnp.int32)
                    u = lax.bitcast_convert_type(
                        jnp.int32(0x5F3759DF) - (bits >> 1), jnp.float32)
                    u = u * (1.5 - 0.5 * var * u * u)
                    mr = mean * u
                    tvv = jnp.full((L,), t, jnp.int32)
                    for k in range(nv):
                        o = (h[k] * u - mr) * gvec[k] + bvec[k]
                        plsc.store_scatter(
                            ob_v.at[_slab], [crows[k], tvv], o)
                return carry

            lax.fori_loop(0, BPW // 8, tok4, 0)

        fire_gather(0, 0)

        def stage_pair(j, carry):
            for slab in range(2):
                s = 2 * j + slab
                wait_gather(slab)

                @pl.when(s + 1 < S)
                def _():
                    fire_gather(s + 1, 1 - slab)

                @pl.when(s >= 1)
                def _():
                    wait_outs(1 - slab)

                compute(s, slab)
                fire_outs(s, slab)
            return carry

        lax.fori_loop(0, S // 2, stage_pair, 0)
        wait_outs(1)

    return body(table_p, x, pos8, gamma, beta)


def kernel(x, table, gamma, beta, pos_embed):
    b, s = x.shape
    xi = x.astype(jnp.int32)
    pos = lax.slice(pos_embed, (0, 1, 0), (1, s + 1, D))[0]
    pos8 = pos * (1.0 / SQRT_D)
    out5 = _fused_kernel(table, xi, pos8, gamma, beta)
    return out5.transpose(2, 4, 0, 1, 3).reshape(b, s, D)


# speedup vs baseline: 1.0575x; 1.0575x over previous
import functools
import math

import jax
import jax.numpy as jnp
from jax import lax
from jax.experimental import pallas as pl
from jax.experimental.pallas import tpu as pltpu
from jax.experimental.pallas import tpu_sc as plsc

D = 64
EPS = 1e-6
SQRT_D = math.sqrt(D)

NC = 2
NS = 16
NW = NC * NS

L = 16
NQ = D // 8


def _fused_kernel(table_p, x, pos8, gamma, beta):
    B, S = x.shape
    BPW = B // NW
    mesh = plsc.VectorSubcoreMesh(core_axis_name="c", subcore_axis_name="s")

    @functools.partial(
        pl.kernel,
        out_type=jax.ShapeDtypeStruct((S, NQ, NW, 8, BPW), jnp.float32),
        mesh=mesh,
        scratch_types=[
            pltpu.VMEM((S, BPW), jnp.int32),
            pltpu.VMEM((L, S), jnp.int32),
            pltpu.VMEM((2, BPW, 2 * D), jnp.float32),
            pltpu.VMEM((S, D), jnp.float32),
            pltpu.VMEM((2, D), jnp.float32),
            pltpu.VMEM((2, D, BPW + 1), jnp.float32),
            pltpu.SemaphoreType.DMA,
            pltpu.SemaphoreType.DMA,
        ],
        compiler_params=pltpu.CompilerParams(
            use_tc_tiling_on_sc=False, needs_layout_passes=False
        ),
    )
    def body(table_h, x_h, pos_h, gam_h, bet_h, out_h, idxt_v, tmp_v, slab_v,
             pos_v, gb_v, ob_v, gsem, osem):
        wid = lax.axis_index("s") * NC + lax.axis_index("c")
        b0 = wid * BPW
        pltpu.sync_copy(pos_h, pos_v)
        pltpu.sync_copy(gam_h, gb_v.at[0])
        pltpu.sync_copy(bet_h, gb_v.at[1])
        iota = lax.iota(jnp.int32, L)

        for ch in range(BPW // L):
            pltpu.sync_copy(x_h.at[pl.ds(b0 + ch * L, L)], tmp_v)

            def tr_body(s, carry, _ch=ch):
                sv = jnp.full((L,), s, jnp.int32)
                v = plsc.load_gather(tmp_v, [iota, sv])
                plsc.store_scatter(idxt_v, [sv, iota + _ch * L], v)
                return carry

            lax.fori_loop(0, S, tr_body, 0)

        def fire_gather(s, slab):
            pltpu.async_copy(
                table_h.at[idxt_v.at[s]], slab_v.at[slab], gsem)

        def wait_gather(slab):
            pltpu.make_async_copy(
                table_h.at[pl.ds(0, BPW)], slab_v.at[slab], gsem).wait()

        def fire_outs(s, slab):
            for q in range(NQ):
                pltpu.async_copy(
                    ob_v.at[slab, pl.ds(q * 8, 8), pl.ds(0, BPW)],
                    out_h.at[s, q, wid], osem)

        def wait_outs(slab):
            pltpu.make_async_copy(
                table_h.at[pl.ds(0, D)], slab_v.at[0, pl.ds(0, D)], osem
            ).wait()

        inv_d = 1.0 / D
        inv_dm1 = 1.0 / (D - 1)
        nv = D // L
        gvec = [gb_v[0, pl.ds(L * k, L)] for k in range(nv)]
        bvec = [gb_v[1, pl.ds(L * k, L)] for k in range(nv)]
        crows = [iota + L * k for k in range(nv)]

        def compute(s, slab):
            pvec = [pos_v[s, pl.ds(L * k, L)] for k in range(nv)]

            def tok4(ti, carry, _slab=slab, _pvec=pvec):
                toks = []
                for u in range(8):
                    t = ti * 8 + u
                    h = [
                        slab_v[_slab, t, pl.ds(L * k, L)] + _pvec[k]
                        for k in range(nv)
                    ]
                    toks.append((t, h))
                stats = []
                for t, h in toks:
                    hs = (h[0] + h[1]) + (h[2] + h[3])
                    hq = (h[0] * h[0] + h[1] * h[1]) + (
                        h[2] * h[2] + h[3] * h[3])
                    stats.append((jnp.sum(hs), jnp.sum(hq)))
                for (t, h), (sm, sq) in zip(toks, stats):
                    mean = sm * inv_d
                    var = jnp.maximum((sq - sm * mean) * inv_dm1, 1e-30)
                    bits = lax.bitcast_convert_type(var, jnp.int32)
                    u = lax.bitcast_convert_type(
                        jnp.int32(0x5F3759DF) - (bits >> 1), jnp.float32)
                    u = u * (1.5 - 0.5 * var * u * u)
                    mr = mean * u
                    tvv = jnp.full((L,), t, jnp.int32)
                    for k in range(nv):
                        o = (h[k] * u - mr) * gvec[k] + bvec[k]
                        plsc.store_scatter(
                            ob_v.at[_slab], [crows[k], tvv], o)
                return carry

            lax.fori_loop(0, BPW // 8, tok4, 0)

        fire_gather(0, 0)

        def stage_pair(j, carry):
            for slab in range(2):
                s = 2 * j + slab
                wait_gather(slab)

                @pl.when(s + 1 < S)
                def _():
                    fire_gather(s + 1, 1 - slab)

                @pl.when(s >= 1)
                def _():
                    wait_outs(1 - slab)

                compute(s, slab)
                fire_outs(s, slab)
            return carry

        lax.fori_loop(0, S // 2, stage_pair, 0)
        wait_outs(1)

    return body(table_p, x, pos8, gamma, beta)


def kernel(x, table, gamma, beta, pos_embed):
    b, s = x.shape
    xi = x.astype(jnp.int32)
    pos = lax.slice(pos_embed, (0, 1, 0), (1, s + 1, D))[0]
    pos8 = pos * (1.0 / SQRT_D)
    table_p = lax.pad(table, jnp.float32(0), ((0, 0, 0), (0, D, 0)))
    out5 = _fused_kernel(table_p, xi, pos8, gamma, beta)
    return out5.transpose(2, 4, 0, 1, 3).reshape(b, s, D)
